# baseline (speedup 1.0000x reference)
import jax
import jax.numpy as jnp
from jax import lax
from jax.experimental import pallas as pl
from jax.experimental.pallas import tpu as pltpu
from jax.experimental.pallas import tpu_sc as plsc

VOCAB = 1000000
D = 64
B = 4096
W = 20
K = 16
L = 8

NC = 2
NS = 16
NW = NC * NS
NB = B // NW

CPACK = 16
XPACK = 48
NE_CH = 4
NE_CW = (W * K) // NE_CH
WVO = CPACK + XPACK + W * K
PACK = WVO + 16
SLOT = 24
NEG_FILL = -100.0
TCG = 16


def _sc_scores(center_table, context_table, pack):
  mesh = plsc.VectorSubcoreMesh(
      core_axis_name="c", subcore_axis_name="s", num_cores=NC,
      num_subcores=NS)

  def body(center_hbm, context_hbm, pk_hbm, wout_hbm, mout_hbm,
           pk0, pk1, cr0, cr1, xr0, xr1, nr0, nr1, ws0, ws1, ms0, ms1,
           s_pk0, s_pk1, s_g0, s_g1, s_w0, s_w1):
    wid = lax.axis_index("s") * NC + lax.axis_index("c")
    base = wid * NB
    lane = lax.iota(jnp.int32, 16)
    mask15 = lane == 15
    fill = jnp.full((16,), NEG_FILL, jnp.float32)
    PK = (pk0, pk1)
    CR = (cr0, cr1)
    XR = (xr0, xr1)
    NR = (nr0, nr1)
    WS = (ws0, ws1)
    MS = (ms0, ms1)
    S_PK = (s_pk0, s_pk1)
    S_G = (s_g0, s_g1)
    S_W = (s_w0, s_w1)

    for p in range(2):
      WS[p][pl.ds(0, 16)] = fill
      WS[p][pl.ds(8, 16)] = fill
      for t in range(W * SLOT // 16):
        MS[p][pl.ds(16 * t, 16)] = fill

    def fire_pack(b, p):
      pltpu.async_copy(pk_hbm.at[pl.ds(b * PACK, PACK)], PK[p], S_PK[p])

    def drain_pack(p):
      pltpu.make_async_copy(pk_hbm.at[pl.ds(0, PACK)], PK[p], S_PK[p]).wait()

    def gather_list(p):
      yield center_hbm, PK[p].at[pl.ds(0, CPACK)], CR[p]
      yield context_hbm, PK[p].at[pl.ds(CPACK, XPACK)], XR[p]
      for j in range(NE_CH):
        yield (context_hbm, PK[p].at[pl.ds(CPACK + XPACK + j * NE_CW, NE_CW)],
               NR[p].at[pl.ds(j * NE_CW, NE_CW)])

    def fire_gathers(p):
      for tab, idx, dst in gather_list(p):
        pltpu.async_copy(tab.at[idx], dst, S_G[p])

    def drain_gathers(p):
      for tab, idx, dst in gather_list(p):
        pltpu.make_async_copy(tab.at[idx], dst, S_G[p]).wait()

    def fire_writes(b, p):
      pltpu.async_copy(WS[p], wout_hbm.at[pl.ds(b * SLOT, SLOT)], S_W[p])
      pltpu.async_copy(MS[p], mout_hbm.at[pl.ds(b * W * SLOT, W * SLOT)],
                       S_W[p])

    def drain_writes(p):
      pltpu.make_async_copy(WS[p], wout_hbm.at[pl.ds(0, SLOT)], S_W[p]).wait()
      pltpu.make_async_copy(MS[p], mout_hbm.at[pl.ds(0, W * SLOT)],
                            S_W[p]).wait()

    def splat(ref, pos):
      return plsc.load_gather(ref, [jnp.full((16,), pos, jnp.int32)])

    mhi = jnp.full((16,), -65536, jnp.int32)

    def chunks(ref, r):
      out = []
      for t in range(2):
        w = plsc.bitcast(ref[r, pl.ds(32 * t, 32)], jnp.int32)
        out.append(plsc.bitcast(w << 16, jnp.float32))
        out.append(plsc.bitcast(w & mhi, jnp.float32))
      return out

    gdn = jax.lax.GatherDimensionNumbers(
        offset_dims=(), collapsed_slice_dims=(0,), start_index_map=(0,))
    perms = [(lane ^ sh)[:, None] for sh in (8, 4, 2, 1)]

    def dot(ref, r, q):
      c = chunks(ref, r)
      e = c[0] * q[0] + c[1] * q[1] + c[2] * q[2] + c[3] * q[3]
      for pm in perms:
        e = e + jax.lax.gather(
            e, pm, gdn, (1,), mode=jax.lax.GatherScatterMode.PROMISE_IN_BOUNDS)
      return e

    def put(ref, slot, vec):
      plsc.store_scatter(ref, [jnp.full((16,), slot, jnp.int32)], vec,
                         mask=mask15)

    def compute(p):
      qc = chunks(CR[p], 0)
      put(WS[p], 16, -dot(XR[p], 0, qc))
      for k in range(K):
        put(WS[p], k, dot(XR[p], 1 + k, qc))
      qm = [jnp.zeros((16,), jnp.float32) for _ in range(4)]
      for l in range(L):
        wl = plsc.bitcast(splat(PK[p], WVO + l), jnp.float32)
        rc = chunks(CR[p], 1 + l)
        qm = [qm[t] + wl * rc[t] for t in range(4)]

      @pl.loop(0, W)
      def _(w):
        vw = splat(PK[p], CPACK + 1 + K + w)
        mf = (vw != 0).astype(jnp.float32)
        off = (mf - 1.0) * (-NEG_FILL)
        put(MS[p], w * SLOT + 16, -dot(XR[p], 1 + K + w, qm) * mf + off)
        for k in range(K):
          put(MS[p], w * SLOT + k, dot(NR[p], w * K + k, qm) * mf + off)

    fire_pack(base, 0)
    fire_pack(base + 1, 1)
    drain_pack(0)
    fire_gathers(0)

    @pl.loop(0, NB, step=2)
    def _(i):
      for p in range(2):
        ie = i + p
        b = base + ie

        @pl.when(ie + 1 < NB)
        def _():
          drain_pack(1 - p)
          fire_gathers(1 - p)

        drain_gathers(p)

        @pl.when(ie >= 2)
        def _():
          drain_writes(p)

        compute(p)
        fire_writes(b, p)

        @pl.when(ie + 2 < NB)
        def _():
          fire_pack(b + 2, p)

    drain_writes(0)
    drain_writes(1)

  f = pl.kernel(
      body,
      out_type=[jax.ShapeDtypeStruct((B * SLOT,), jnp.float32),
                jax.ShapeDtypeStruct((B * W * SLOT,), jnp.float32)],
      mesh=mesh,
      compiler_params=pltpu.CompilerParams(needs_layout_passes=False,
                                           use_tc_tiling_on_sc=False),
      scratch_types=(
          [pltpu.VMEM((PACK,), jnp.int32)] * 2
          + [pltpu.VMEM((CPACK, D), jnp.bfloat16)] * 2
          + [pltpu.VMEM((XPACK, D), jnp.bfloat16)] * 2
          + [pltpu.VMEM((W * K, D), jnp.bfloat16)] * 2
          + [pltpu.VMEM((SLOT,), jnp.float32)] * 2
          + [pltpu.VMEM((W * SLOT,), jnp.float32)] * 2
          + [pltpu.SemaphoreType.DMA] * 6
      ),
  )
  return f(center_table, context_table, pack)


def _tc_loss(word_sc, mwe_sc, vmask):
  wr = B * SLOT // 128 // TCG
  mr = B * W * SLOT // 128 // TCG
  vr = B * W // 128 // TCG

  def body(w_ref, m_ref, v_ref, o_ref, acc):
    def sp(x):
      return jnp.maximum(x, 0.0) + jnp.log1p(jnp.exp(-jnp.abs(x)))

    i = pl.program_id(0)

    @pl.when(i == 0)
    def _():
      acc[0] = 0.0
      acc[1] = 0.0
      acc[2] = 0.0

    acc[0] += jnp.sum(sp(w_ref[...]))
    acc[1] += jnp.sum(sp(m_ref[...]))
    acc[2] += jnp.sum(v_ref[...])

    @pl.when(i == TCG - 1)
    def _():
      lw = acc[0] / B
      lm = acc[1] / jnp.maximum(acc[2], 1.0)
      o_ref[...] = jnp.full((1, 1), lw + 25.0 * lm, jnp.float32)

  out = pl.pallas_call(
      body,
      grid=(TCG,),
      in_specs=[pl.BlockSpec((wr, 128), lambda i: (i, 0)),
                pl.BlockSpec((mr, 128), lambda i: (i, 0)),
                pl.BlockSpec((vr, 128), lambda i: (i, 0))],
      out_specs=pl.BlockSpec((1, 1), lambda i: (0, 0)),
      out_shape=jax.ShapeDtypeStruct((1, 1), jnp.float32),
      scratch_shapes=[pltpu.SMEM((3,), jnp.float32)],
  )(word_sc, mwe_sc, vmask)
  return out[0, 0]


def kernel(center_words, outside_words, negative_examples_words, mwe_words,
           mwe_length, outside_mwe_words, negative_examples_mwe,
           center_table, context_table):
  i32 = jnp.int32
  zc = jnp.zeros((B, CPACK - 1 - L), i32)
  zx = jnp.zeros((B, XPACK - 1 - K - W), i32)
  lenf = mwe_length.astype(jnp.float32)[:, None]
  wv = (jnp.arange(L)[None, :] < mwe_length[:, None]).astype(jnp.float32) / lenf
  wv = jnp.concatenate([wv, jnp.zeros((B, 16 - L), jnp.float32)], axis=1)
  pack = jnp.concatenate(
      [center_words[:, None].astype(i32), mwe_words.astype(i32), zc,
       outside_words[:, None].astype(i32),
       negative_examples_words.astype(i32),
       outside_mwe_words.astype(i32), zx,
       negative_examples_mwe.astype(i32).reshape(B, W * K),
       lax.bitcast_convert_type(wv, i32)], axis=1).reshape(-1)
  vmask = (outside_mwe_words.reshape(-1) != 0).astype(jnp.float32)

  word_sc, mwe_sc = _sc_scores(center_table.astype(jnp.bfloat16),
                               context_table.astype(jnp.bfloat16), pack)
  return _tc_loss(word_sc.reshape(B * SLOT // 128, 128),
                  mwe_sc.reshape(B * W * SLOT // 128, 128),
                  vmask.reshape(B * W // 128, 128))


# speedup vs baseline: 1.1786x; 1.1786x over previous
import jax
import jax.numpy as jnp
from jax import lax
from jax.experimental import pallas as pl
from jax.experimental.pallas import tpu as pltpu
from jax.experimental.pallas import tpu_sc as plsc

VOCAB = 1000000
D = 64
B = 4096
W = 20
K = 16
L = 8

NC = 2
NS = 16
NW = NC * NS
NB = B // NW

CPACK = 16
XPACK = 48
NE_CH = 4
NE_CW = (W * K) // NE_CH
WVO = CPACK + XPACK + W * K
PACK = WVO + 16
SLOT = 24
NEG_FILL = -100.0
TCG = 16


def _sc_scores(center_table, context_table, pack):
  mesh = plsc.VectorSubcoreMesh(
      core_axis_name="c", subcore_axis_name="s", num_cores=NC,
      num_subcores=NS)

  def body(center_hbm, context_hbm, pk_hbm, wout_hbm, mout_hbm,
           pk0, pk1, cr0, cr1, xr0, xr1, nr0, nr1, ws0, ws1, ms0, ms1,
           s_pk0, s_pk1, s_g0, s_g1, s_w0, s_w1):
    wid = lax.axis_index("s") * NC + lax.axis_index("c")
    base = wid * NB
    lane = lax.iota(jnp.int32, 16)
    mask15 = lane == 15
    fill = jnp.full((16,), NEG_FILL, jnp.float32)
    PK = (pk0, pk1)
    CR = (cr0, cr1)
    XR = (xr0, xr1)
    NR = (nr0, nr1)
    WS = (ws0, ws1)
    MS = (ms0, ms1)
    S_PK = (s_pk0, s_pk1)
    S_G = (s_g0, s_g1)
    S_W = (s_w0, s_w1)

    for p in range(2):
      WS[p][pl.ds(0, 16)] = fill
      WS[p][pl.ds(8, 16)] = fill
      for t in range(W * SLOT // 16):
        MS[p][pl.ds(16 * t, 16)] = fill

    def fire_pack(b, p):
      pltpu.async_copy(pk_hbm.at[pl.ds(b * PACK, PACK)], PK[p], S_PK[p])

    def drain_pack(p):
      pltpu.make_async_copy(pk_hbm.at[pl.ds(0, PACK)], PK[p], S_PK[p]).wait()

    def gather_list(p):
      yield center_hbm, PK[p].at[pl.ds(0, 1 + L)], CR[p]
      yield context_hbm, PK[p].at[pl.ds(CPACK, 1 + K + W)], XR[p]
      for j in range(NE_CH):
        yield (context_hbm, PK[p].at[pl.ds(CPACK + XPACK + j * NE_CW, NE_CW)],
               NR[p].at[pl.ds(j * NE_CW, NE_CW)])

    def fire_gathers(p):
      for tab, idx, dst in gather_list(p):
        pltpu.async_copy(tab.at[idx], dst, S_G[p])

    def drain_gathers(p):
      for tab, idx, dst in gather_list(p):
        pltpu.make_async_copy(tab.at[idx], dst, S_G[p]).wait()

    def fire_writes(b, p):
      pltpu.async_copy(WS[p], wout_hbm.at[pl.ds(b * SLOT, SLOT)], S_W[p])
      pltpu.async_copy(MS[p], mout_hbm.at[pl.ds(b * W * SLOT, W * SLOT)],
                       S_W[p])

    def drain_writes(p):
      pltpu.make_async_copy(WS[p], wout_hbm.at[pl.ds(0, SLOT)], S_W[p]).wait()
      pltpu.make_async_copy(MS[p], mout_hbm.at[pl.ds(0, W * SLOT)],
                            S_W[p]).wait()

    def splat(ref, pos):
      return plsc.load_gather(ref, [jnp.full((16,), pos, jnp.int32)])

    def chunks(ref, r):
      return [ref[r, pl.ds(16 * t, 16)] for t in range(4)]

    gdn = jax.lax.GatherDimensionNumbers(
        offset_dims=(), collapsed_slice_dims=(0,), start_index_map=(0,))
    perms = [(lane ^ sh)[:, None] for sh in (8, 4, 2, 1)]

    def dot(ref, r, q):
      c = chunks(ref, r)
      e = c[0] * q[0] + c[1] * q[1] + c[2] * q[2] + c[3] * q[3]
      for pm in perms:
        e = e + jax.lax.gather(
            e, pm, gdn, (1,), mode=jax.lax.GatherScatterMode.PROMISE_IN_BOUNDS)
      return e

    def put(ref, slot, vec):
      plsc.store_scatter(ref, [jnp.full((16,), slot, jnp.int32)], vec,
                         mask=mask15)

    def compute(p):
      qc = chunks(CR[p], 0)
      put(WS[p], 16, -dot(XR[p], 0, qc))
      for k in range(K):
        put(WS[p], k, dot(XR[p], 1 + k, qc))
      qm = [jnp.zeros((16,), jnp.float32) for _ in range(4)]
      for l in range(L):
        wl = plsc.bitcast(splat(PK[p], WVO + l), jnp.float32)
        rc = chunks(CR[p], 1 + l)
        qm = [qm[t] + wl * rc[t] for t in range(4)]

      @pl.loop(0, W)
      def _(w):
        vw = splat(PK[p], CPACK + 1 + K + w)
        mf = (vw != 0).astype(jnp.float32)
        off = (mf - 1.0) * (-NEG_FILL)
        put(MS[p], w * SLOT + 16, -dot(XR[p], 1 + K + w, qm) * mf + off)
        for k in range(K):
          put(MS[p], w * SLOT + k, dot(NR[p], w * K + k, qm) * mf + off)

    fire_pack(base, 0)
    fire_pack(base + 1, 1)
    drain_pack(0)
    fire_gathers(0)

    @pl.loop(0, NB, step=2)
    def _(i):
      for p in range(2):
        ie = i + p
        b = base + ie

        @pl.when(ie + 1 < NB)
        def _():
          drain_pack(1 - p)
          fire_gathers(1 - p)

        drain_gathers(p)

        @pl.when(ie >= 2)
        def _():
          drain_writes(p)

        compute(p)
        fire_writes(b, p)

        @pl.when(ie + 2 < NB)
        def _():
          fire_pack(b + 2, p)

    drain_writes(0)
    drain_writes(1)

  f = pl.kernel(
      body,
      out_type=[jax.ShapeDtypeStruct((B * SLOT,), jnp.float32),
                jax.ShapeDtypeStruct((B * W * SLOT,), jnp.float32)],
      mesh=mesh,
      compiler_params=pltpu.CompilerParams(needs_layout_passes=False,
                                           use_tc_tiling_on_sc=False),
      scratch_types=(
          [pltpu.VMEM((PACK,), jnp.int32)] * 2
          + [pltpu.VMEM((1 + L, D), jnp.float32)] * 2
          + [pltpu.VMEM((1 + K + W, D), jnp.float32)] * 2
          + [pltpu.VMEM((W * K, D), jnp.float32)] * 2
          + [pltpu.VMEM((SLOT,), jnp.float32)] * 2
          + [pltpu.VMEM((W * SLOT,), jnp.float32)] * 2
          + [pltpu.SemaphoreType.DMA] * 6
      ),
  )
  return f(center_table, context_table, pack)


def _tc_loss(word_sc, mwe_sc, vmask):
  wr = B * SLOT // 128 // TCG
  mr = B * W * SLOT // 128 // TCG
  vr = B * W // 128 // TCG

  def body(w_ref, m_ref, v_ref, o_ref, acc):
    def sp(x):
      return jnp.maximum(x, 0.0) + jnp.log1p(jnp.exp(-jnp.abs(x)))

    i = pl.program_id(0)

    @pl.when(i == 0)
    def _():
      acc[0] = 0.0
      acc[1] = 0.0
      acc[2] = 0.0

    acc[0] += jnp.sum(sp(w_ref[...]))
    acc[1] += jnp.sum(sp(m_ref[...]))
    acc[2] += jnp.sum(v_ref[...])

    @pl.when(i == TCG - 1)
    def _():
      lw = acc[0] / B
      lm = acc[1] / jnp.maximum(acc[2], 1.0)
      o_ref[...] = jnp.full((1, 1), lw + 25.0 * lm, jnp.float32)

  out = pl.pallas_call(
      body,
      grid=(TCG,),
      in_specs=[pl.BlockSpec((wr, 128), lambda i: (i, 0)),
                pl.BlockSpec((mr, 128), lambda i: (i, 0)),
                pl.BlockSpec((vr, 128), lambda i: (i, 0))],
      out_specs=pl.BlockSpec((1, 1), lambda i: (0, 0)),
      out_shape=jax.ShapeDtypeStruct((1, 1), jnp.float32),
      scratch_shapes=[pltpu.SMEM((3,), jnp.float32)],
  )(word_sc, mwe_sc, vmask)
  return out[0, 0]


def kernel(center_words, outside_words, negative_examples_words, mwe_words,
           mwe_length, outside_mwe_words, negative_examples_mwe,
           center_table, context_table):
  i32 = jnp.int32
  zc = jnp.zeros((B, CPACK - 1 - L), i32)
  zx = jnp.zeros((B, XPACK - 1 - K - W), i32)
  lenf = mwe_length.astype(jnp.float32)[:, None]
  wv = (jnp.arange(L)[None, :] < mwe_length[:, None]).astype(jnp.float32) / lenf
  wv = jnp.concatenate([wv, jnp.zeros((B, 16 - L), jnp.float32)], axis=1)
  pack = jnp.concatenate(
      [center_words[:, None].astype(i32), mwe_words.astype(i32), zc,
       outside_words[:, None].astype(i32),
       negative_examples_words.astype(i32),
       outside_mwe_words.astype(i32), zx,
       negative_examples_mwe.astype(i32).reshape(B, W * K),
       lax.bitcast_convert_type(wv, i32)], axis=1).reshape(-1)
  vmask = (outside_mwe_words.reshape(-1) != 0).astype(jnp.float32)

  word_sc, mwe_sc = _sc_scores(center_table, context_table, pack)
  return _tc_loss(word_sc.reshape(B * SLOT // 128, 128),
                  mwe_sc.reshape(B * W * SLOT // 128, 128),
                  vmask.reshape(B * W // 128, 128))
